# baseline (speedup 1.0000x reference)
import jax
import jax.numpy as jnp
from jax.experimental import pallas as pl


def _copy_body(x_ref, o_ref):
    o_ref[...] = x_ref[...]


def _gcn(h, src, dst, W, b, n):
    h = h @ W
    deg = jnp.zeros((n,), h.dtype).at[dst].add(1.0)
    dinv = jnp.where(deg > 0, 1.0 / jnp.sqrt(deg), 0.0)
    norm = dinv[src] * dinv[dst]
    out = jnp.zeros_like(h).at[dst].add(h[src] * norm[:, None])
    return out + b


def kernel(x, edge_index, W0, b0, W1, b1, W2, b2, W3, b3, W4, b4):
    B, N, T = x.shape
    H = 10
    n_tot = B * N
    src = jnp.concatenate([edge_index[0] + bi * N for bi in range(B)])
    dst = jnp.concatenate([edge_index[1] + bi * N for bi in range(B)])
    loops = jnp.arange(n_tot, dtype=src.dtype)
    src = jnp.concatenate([src, loops])
    dst = jnp.concatenate([dst, loops])
    Ws = [W0, W1, W2, W3, W4]
    bs = [b0, b1, b2, b3, b4]

    def f(y):
        h = y
        for i in range(5):
            h = _gcn(h, src, dst, Ws[i], bs[i], n_tot)
            if i < 4:
                h = jnp.tanh(h)
        return h

    y = x[:, :, -1].reshape(n_tot, 1)
    y = pl.pallas_call(
        _copy_body, out_shape=jax.ShapeDtypeStruct(y.shape, y.dtype)
    )(y)
    t = jnp.linspace(0.0, float(H), H)
    ys = [y]
    for s in range(H - 1):
        dt = t[s + 1] - t[s]
        k1 = f(y)
        k2 = f(y + dt * k1 / 3.0)
        k3 = f(y + dt * (k2 - k1 / 3.0))
        k4 = f(y + dt * (k1 - k2 + k3))
        y = y + dt * (k1 + 3.0 * k2 + 3.0 * k3 + k4) / 8.0
        ys.append(y)
    preds = jnp.stack(ys).reshape(H, B, N)
    preds = jnp.transpose(preds, (1, 2, 0))
    return preds


# speedup vs baseline: 11.1368x; 11.1368x over previous
import functools

import jax
import jax.numpy as jnp
from jax import lax
from jax.experimental import pallas as pl
from jax.experimental.pallas import tpu as pltpu
from jax.experimental.pallas import tpu_sc as plsc

_B = 2
_N = 10000
_NT = _B * _N
_E = 160000
_EB = _E + _N
_H = 10
_DT = 10.0 / 9.0

_NSC = 16
_CK = 128
_CHT = 84
_EPT = _CHT * _CK
_NBIN = _NSC * 632
_SPT = 632
_NTP = _B * _NBIN

_R = 2528
_G = _NTP // _R


def _make_prop(width):
    mesh = plsc.VectorSubcoreMesh(core_axis_name="c", subcore_axis_name="s")

    @functools.partial(
        pl.kernel,
        out_type=jax.ShapeDtypeStruct((_NTP, width), jnp.float32),
        mesh=mesh,
        scratch_types=[
            pltpu.VMEM((_CHT, _CK), jnp.int32),
            pltpu.VMEM((_CHT, _CK), jnp.int32),
            pltpu.VMEM((_CK, width), jnp.float32),
            pltpu.VMEM_SHARED((_NBIN, width), jnp.float32),
            pltpu.SemaphoreType.DMA,
        ],
        compiler_params=pltpu.CompilerParams(use_tc_tiling_on_sc=False),
    )
    def prop(g_hbm, src_hbm, dst_hbm, zeros_hbm, out_hbm,
             srcv, dstv, rows, acc, sem):
        c = lax.axis_index("c")
        s = lax.axis_index("s")
        pltpu.sync_copy(src_hbm.at[c, s], srcv)
        pltpu.sync_copy(dst_hbm.at[c, s], dstv)
        z0 = s * _SPT
        pltpu.sync_copy(zeros_hbm.at[pl.ds(z0, _SPT)], acc.at[pl.ds(z0, _SPT)])
        plsc.subcore_barrier()

        def body(j, carry):
            pltpu.async_copy(g_hbm.at[srcv.at[j]], rows, sem).wait()
            pltpu.sync_copy(rows, acc.at[dstv.at[j]], add=True)
            return carry

        lax.fori_loop(0, _CHT, body, 0)
        plsc.subcore_barrier()
        r0 = s * _SPT
        pltpu.sync_copy(acc.at[pl.ds(r0, _SPT)],
                        out_hbm.at[pl.ds(c * _NBIN + r0, _SPT)])

    return prop


_prop64 = _make_prop(64)
_prop8 = _make_prop(8)


def _wide_spec():
    return pl.BlockSpec((_R, 64), lambda i: (i, 0))


def _narrow_spec():
    return pl.BlockSpec((_R, 8), lambda i: (i, 0))


def _full_spec(shape):
    return pl.BlockSpec(shape, lambda i: tuple(0 for _ in shape))


def _dinv_body(deg_ref, o_ref):
    d = deg_ref[...]
    o_ref[...] = jnp.where(d > 0.0, lax.rsqrt(d), 0.0)


def _tc_dinv(deg8):
    return pl.pallas_call(
        _dinv_body,
        grid=(_G,),
        in_specs=[_narrow_spec()],
        out_specs=_narrow_spec(),
        out_shape=jax.ShapeDtypeStruct((_NTP, 8), jnp.float32),
    )(deg8)


def _l1_body(s_ref, d_ref, w0_ref, b0_ref, w1_ref, o_ref):
    dcol = d_ref[:, 0:1]
    z = s_ref[:, 0:1] * dcol
    h = jnp.tanh(z * w0_ref[...] + b0_ref[...])
    o_ref[...] = jnp.dot(h, w1_ref[...], precision=lax.Precision.HIGHEST,
                         preferred_element_type=jnp.float32) * dcol


def _tc_l1(s08, dinv8, w0, b0r, w1):
    return pl.pallas_call(
        _l1_body,
        grid=(_G,),
        in_specs=[_narrow_spec(), _narrow_spec(), _full_spec((1, 64)),
                  _full_spec((1, 64)), _full_spec((64, 64))],
        out_specs=_wide_spec(),
        out_shape=jax.ShapeDtypeStruct((_NTP, 64), jnp.float32),
    )(s08, dinv8, w0, b0r, w1)


def _mid_body(s_ref, d_ref, bp_ref, w_ref, o_ref):
    dcol = d_ref[:, 0:1]
    h = jnp.tanh(s_ref[...] * dcol + bp_ref[...])
    o_ref[...] = jnp.dot(h, w_ref[...], precision=lax.Precision.HIGHEST,
                         preferred_element_type=jnp.float32) * dcol


def _tc_mid(s, dinv8, bpr, w):
    return pl.pallas_call(
        _mid_body,
        grid=(_G,),
        in_specs=[_wide_spec(), _narrow_spec(), _full_spec((1, 64)),
                  _full_spec((64, 64))],
        out_specs=_wide_spec(),
        out_shape=jax.ShapeDtypeStruct((_NTP, 64), jnp.float32),
    )(s, dinv8, bpr, w)


def _l4_body(s_ref, d_ref, b3_ref, w4_ref, e0_ref, o_ref):
    dcol = d_ref[:, 0:1]
    h = jnp.tanh(s_ref[...] * dcol + b3_ref[...])
    v = jnp.dot(h, w4_ref[...], precision=lax.Precision.HIGHEST,
                preferred_element_type=jnp.float32) * dcol
    o_ref[...] = v * e0_ref[...]


def _tc_l4(s3, dinv8, b3r, w4, e0):
    return pl.pallas_call(
        _l4_body,
        grid=(_G,),
        in_specs=[_wide_spec(), _narrow_spec(), _full_spec((1, 64)),
                  _full_spec((64, 1)), _full_spec((1, 8))],
        out_specs=_narrow_spec(),
        out_shape=jax.ShapeDtypeStruct((_NTP, 8), jnp.float32),
    )(s3, dinv8, b3r, w4, e0)


def _make_combo(coefs):
    nk = len(coefs)

    def body(*refs):
        y_ref, d_ref, be_ref = refs[0], refs[1], refs[2]
        s_refs = refs[3:3 + nk]
        y_o, g_o = refs[3 + nk], refs[4 + nk]
        d = d_ref[...]
        acc = y_ref[...]
        for a, sref in zip(coefs, s_refs):
            acc = acc + a * (d * sref[...] + be_ref[...])
        y_o[...] = acc
        g_o[...] = acc * d

    def call(y8, dinv8, b4e0, *ss):
        return pl.pallas_call(
            body,
            grid=(_G,),
            in_specs=[_narrow_spec(), _narrow_spec(), _full_spec((1, 8))]
            + [_narrow_spec()] * nk,
            out_specs=(_narrow_spec(), _narrow_spec()),
            out_shape=(jax.ShapeDtypeStruct((_NTP, 8), jnp.float32),
                       jax.ShapeDtypeStruct((_NTP, 8), jnp.float32)),
        )(y8, dinv8, b4e0, *ss)

    return call


_combo0 = _make_combo(())
_combo1 = _make_combo((_DT / 3.0,))
_combo2 = _make_combo((-_DT / 3.0, _DT))
_combo3 = _make_combo((_DT, -_DT, _DT))
_combo4 = _make_combo((_DT / 8.0, 3.0 * _DT / 8.0, 3.0 * _DT / 8.0, _DT / 8.0))


def kernel(x, edge_index, W0, b0, W1, b1, W2, b2, W3, b3, W4, b4):
    f32 = jnp.float32
    loops = jnp.arange(_N, dtype=jnp.int32)
    srcb = jnp.concatenate([edge_index[0].astype(jnp.int32), loops])
    dstb = jnp.concatenate([edge_index[1].astype(jnp.int32), loops])
    pad = _NSC * _EPT - _EB
    srcp = jnp.concatenate([srcb, jnp.zeros((pad,), jnp.int32)])
    dstp = jnp.concatenate([dstb, jnp.full((pad,), _N, jnp.int32)])
    src_t = jnp.stack([srcp.reshape(_NSC, _CHT, _CK),
                       (srcp + _NBIN).reshape(_NSC, _CHT, _CK)])
    dst_loc = dstp.reshape(_NSC, _CHT, _CK)
    dst_t = jnp.stack([dst_loc, dst_loc])

    zeros64 = jnp.zeros((_NBIN, 64), f32)
    zeros8 = jnp.zeros((_NBIN, 8), f32)
    e0 = jnp.zeros((1, 8), f32).at[0, 0].set(1.0)
    ones8 = jnp.tile(e0, (_NTP, 1))

    b0r = b0.reshape(1, 64)
    b1r = b1.reshape(1, 64)
    b2r = b2.reshape(1, 64)
    b3r = b3.reshape(1, 64)
    w0 = W0.reshape(1, 64)
    b4e0 = b4.reshape(1, 1) * e0

    deg8 = _prop8(ones8, src_t, dst_t, zeros8)
    dinv8 = _tc_dinv(deg8)

    y0 = x[:, :, -1].astype(f32)
    y0p = jnp.pad(y0, ((0, 0), (0, _NBIN - _N))).reshape(_NTP, 1)
    y0_8 = jnp.pad(y0p, ((0, 0), (0, 7)))
    y8, g8 = _combo0(y0_8, dinv8, b4e0)

    def feval(g08):
        s0 = _prop8(g08, src_t, dst_t, zeros8)
        g1 = _tc_l1(s0, dinv8, w0, b0r, W1)
        s1 = _prop64(g1, src_t, dst_t, zeros64)
        g2 = _tc_mid(s1, dinv8, b1r, W2)
        s2 = _prop64(g2, src_t, dst_t, zeros64)
        g3 = _tc_mid(s2, dinv8, b2r, W3)
        s3 = _prop64(g3, src_t, dst_t, zeros64)
        return _tc_l4(s3, dinv8, b3r, W4, e0)

    def step(carry, _):
        y, g = carry
        sk1 = feval(g)
        y2, g2s = _combo1(y, dinv8, b4e0, sk1)
        sk2 = feval(g2s)
        y3, g3s = _combo2(y, dinv8, b4e0, sk1, sk2)
        sk3 = feval(g3s)
        y4, g4s = _combo3(y, dinv8, b4e0, sk1, sk2, sk3)
        sk4 = feval(g4s)
        yn, gn = _combo4(y, dinv8, b4e0, sk1, sk2, sk3, sk4)
        return (yn, gn), yn[:, 0]

    (_, _), ys = lax.scan(step, (y8, g8), None, length=_H - 1)

    preds = jnp.concatenate([y0_8[None, :, 0], ys], axis=0)
    preds = preds.reshape(_H, _B, _NBIN)[:, :, :_N]
    return jnp.transpose(preds, (1, 2, 0))


# speedup vs baseline: 15.4281x; 1.3853x over previous
import functools

import jax
import jax.numpy as jnp
from jax import lax
from jax.experimental import pallas as pl
from jax.experimental.pallas import tpu as pltpu
from jax.experimental.pallas import tpu_sc as plsc

_B = 2
_N = 10000
_NT = _B * _N
_E = 160000
_EB = _E + _N
_H = 10
_DT = 10.0 / 9.0

_NSC = 16
_CK = 128
_CHT = 84
_EPT = _CHT * _CK
_NBIN = _NSC * 632
_SPT = 632
_NTP = _B * _NBIN

_R = 2528
_G = _NTP // _R


def _make_prop(width):
    mesh = plsc.VectorSubcoreMesh(core_axis_name="c", subcore_axis_name="s")
    nb = 4
    ng = _CHT // nb

    @functools.partial(
        pl.kernel,
        out_type=jax.ShapeDtypeStruct((_NTP, width), jnp.float32),
        mesh=mesh,
        scratch_types=[
            pltpu.VMEM((_CHT, _CK), jnp.int32),
            pltpu.VMEM((_CHT, _CK), jnp.int32),
            pltpu.VMEM((nb, _CK, width), jnp.float32),
            pltpu.VMEM_SHARED((_NBIN, width), jnp.float32),
            pltpu.SemaphoreType.DMA((nb,)),
            pltpu.SemaphoreType.DMA((nb,)),
        ],
        compiler_params=pltpu.CompilerParams(use_tc_tiling_on_sc=False),
    )
    def prop(g_hbm, src_hbm, dst_hbm, zeros_hbm, out_hbm,
             srcv, dstv, rows, acc, gsem, ssem):
        c = lax.axis_index("c")
        s = lax.axis_index("s")
        pltpu.sync_copy(src_hbm.at[c, s], srcv)
        pltpu.sync_copy(dst_hbm.at[c, s], dstv)
        z0 = s * _SPT
        pltpu.sync_copy(zeros_hbm.at[pl.ds(z0, _SPT)], acc.at[pl.ds(z0, _SPT)])
        plsc.subcore_barrier()

        for b in range(nb):
            pltpu.async_copy(g_hbm.at[srcv.at[b]], rows.at[b], gsem.at[b])

        def body(g, carry):
            for b in range(nb):
                j = g * nb + b
                pltpu.make_async_copy(g_hbm.at[srcv.at[j]], rows.at[b],
                                      gsem.at[b]).wait()
                pltpu.async_copy(rows.at[b], acc.at[dstv.at[j]], ssem.at[b],
                                 add=True)
            for b in range(nb):
                j = g * nb + b
                pltpu.make_async_copy(rows.at[b], acc.at[dstv.at[j]],
                                      ssem.at[b]).wait()
                jn = j + nb

                @pl.when(jn < _CHT)
                def _():
                    pltpu.async_copy(g_hbm.at[srcv.at[jn]], rows.at[b],
                                     gsem.at[b])
            return carry

        lax.fori_loop(0, ng, body, 0)
        plsc.subcore_barrier()
        r0 = s * _SPT
        pltpu.sync_copy(acc.at[pl.ds(r0, _SPT)],
                        out_hbm.at[pl.ds(c * _NBIN + r0, _SPT)])

    return prop


_prop64 = _make_prop(64)
_prop8 = _make_prop(8)


def _wide_spec():
    return pl.BlockSpec((_R, 64), lambda i: (i, 0))


def _narrow_spec():
    return pl.BlockSpec((_R, 8), lambda i: (i, 0))


def _full_spec(shape):
    return pl.BlockSpec(shape, lambda i: tuple(0 for _ in shape))


def _dinv_body(deg_ref, o_ref):
    d = deg_ref[...]
    o_ref[...] = jnp.where(d > 0.0, lax.rsqrt(d), 0.0)


def _tc_dinv(deg8):
    return pl.pallas_call(
        _dinv_body,
        grid=(_G,),
        in_specs=[_narrow_spec()],
        out_specs=_narrow_spec(),
        out_shape=jax.ShapeDtypeStruct((_NTP, 8), jnp.float32),
    )(deg8)


def _l1_body(s_ref, d_ref, w0_ref, b0_ref, w1_ref, o_ref):
    dcol = d_ref[:, 0:1]
    z = s_ref[:, 0:1] * dcol
    h = jnp.tanh(z * w0_ref[...] + b0_ref[...])
    o_ref[...] = jnp.dot(h, w1_ref[...], precision=lax.Precision.HIGHEST,
                         preferred_element_type=jnp.float32) * dcol


def _tc_l1(s08, dinv8, w0, b0r, w1):
    return pl.pallas_call(
        _l1_body,
        grid=(_G,),
        in_specs=[_narrow_spec(), _narrow_spec(), _full_spec((1, 64)),
                  _full_spec((1, 64)), _full_spec((64, 64))],
        out_specs=_wide_spec(),
        out_shape=jax.ShapeDtypeStruct((_NTP, 64), jnp.float32),
    )(s08, dinv8, w0, b0r, w1)


def _mid_body(s_ref, d_ref, bp_ref, w_ref, o_ref):
    dcol = d_ref[:, 0:1]
    h = jnp.tanh(s_ref[...] * dcol + bp_ref[...])
    o_ref[...] = jnp.dot(h, w_ref[...], precision=lax.Precision.HIGHEST,
                         preferred_element_type=jnp.float32) * dcol


def _tc_mid(s, dinv8, bpr, w):
    return pl.pallas_call(
        _mid_body,
        grid=(_G,),
        in_specs=[_wide_spec(), _narrow_spec(), _full_spec((1, 64)),
                  _full_spec((64, 64))],
        out_specs=_wide_spec(),
        out_shape=jax.ShapeDtypeStruct((_NTP, 64), jnp.float32),
    )(s, dinv8, bpr, w)


def _l4_body(s_ref, d_ref, b3_ref, w4_ref, e0_ref, o_ref):
    dcol = d_ref[:, 0:1]
    h = jnp.tanh(s_ref[...] * dcol + b3_ref[...])
    v = jnp.dot(h, w4_ref[...], precision=lax.Precision.HIGHEST,
                preferred_element_type=jnp.float32) * dcol
    o_ref[...] = v * e0_ref[...]


def _tc_l4(s3, dinv8, b3r, w4, e0):
    return pl.pallas_call(
        _l4_body,
        grid=(_G,),
        in_specs=[_wide_spec(), _narrow_spec(), _full_spec((1, 64)),
                  _full_spec((64, 1)), _full_spec((1, 8))],
        out_specs=_narrow_spec(),
        out_shape=jax.ShapeDtypeStruct((_NTP, 8), jnp.float32),
    )(s3, dinv8, b3r, w4, e0)


def _make_combo(coefs):
    nk = len(coefs)

    def body(*refs):
        y_ref, d_ref, be_ref = refs[0], refs[1], refs[2]
        s_refs = refs[3:3 + nk]
        y_o, g_o = refs[3 + nk], refs[4 + nk]
        d = d_ref[...]
        acc = y_ref[...]
        for a, sref in zip(coefs, s_refs):
            acc = acc + a * (d * sref[...] + be_ref[...])
        y_o[...] = acc
        g_o[...] = acc * d

    def call(y8, dinv8, b4e0, *ss):
        return pl.pallas_call(
            body,
            grid=(_G,),
            in_specs=[_narrow_spec(), _narrow_spec(), _full_spec((1, 8))]
            + [_narrow_spec()] * nk,
            out_specs=(_narrow_spec(), _narrow_spec()),
            out_shape=(jax.ShapeDtypeStruct((_NTP, 8), jnp.float32),
                       jax.ShapeDtypeStruct((_NTP, 8), jnp.float32)),
        )(y8, dinv8, b4e0, *ss)

    return call


_combo0 = _make_combo(())
_combo1 = _make_combo((_DT / 3.0,))
_combo2 = _make_combo((-_DT / 3.0, _DT))
_combo3 = _make_combo((_DT, -_DT, _DT))
_combo4 = _make_combo((_DT / 8.0, 3.0 * _DT / 8.0, 3.0 * _DT / 8.0, _DT / 8.0))


def kernel(x, edge_index, W0, b0, W1, b1, W2, b2, W3, b3, W4, b4):
    f32 = jnp.float32
    loops = jnp.arange(_N, dtype=jnp.int32)
    srcb = jnp.concatenate([edge_index[0].astype(jnp.int32), loops])
    dstb = jnp.concatenate([edge_index[1].astype(jnp.int32), loops])
    pad = _NSC * _EPT - _EB
    srcp = jnp.concatenate([srcb, jnp.zeros((pad,), jnp.int32)])
    dstp = jnp.concatenate([dstb, jnp.full((pad,), _N, jnp.int32)])
    src_t = jnp.stack([srcp.reshape(_NSC, _CHT, _CK),
                       (srcp + _NBIN).reshape(_NSC, _CHT, _CK)])
    dst_loc = dstp.reshape(_NSC, _CHT, _CK)
    dst_t = jnp.stack([dst_loc, dst_loc])

    zeros64 = jnp.zeros((_NBIN, 64), f32)
    zeros8 = jnp.zeros((_NBIN, 8), f32)
    e0 = jnp.zeros((1, 8), f32).at[0, 0].set(1.0)
    ones8 = jnp.tile(e0, (_NTP, 1))

    b0r = b0.reshape(1, 64)
    b1r = b1.reshape(1, 64)
    b2r = b2.reshape(1, 64)
    b3r = b3.reshape(1, 64)
    w0 = W0.reshape(1, 64)
    b4e0 = b4.reshape(1, 1) * e0

    deg8 = _prop8(ones8, src_t, dst_t, zeros8)
    dinv8 = _tc_dinv(deg8)

    y0 = x[:, :, -1].astype(f32)
    y0p = jnp.pad(y0, ((0, 0), (0, _NBIN - _N))).reshape(_NTP, 1)
    y0_8 = jnp.pad(y0p, ((0, 0), (0, 7)))
    y8, g8 = _combo0(y0_8, dinv8, b4e0)

    def feval(g08):
        s0 = _prop8(g08, src_t, dst_t, zeros8)
        g1 = _tc_l1(s0, dinv8, w0, b0r, W1)
        s1 = _prop64(g1, src_t, dst_t, zeros64)
        g2 = _tc_mid(s1, dinv8, b1r, W2)
        s2 = _prop64(g2, src_t, dst_t, zeros64)
        g3 = _tc_mid(s2, dinv8, b2r, W3)
        s3 = _prop64(g3, src_t, dst_t, zeros64)
        return _tc_l4(s3, dinv8, b3r, W4, e0)

    def step(carry, _):
        y, g = carry
        sk1 = feval(g)
        y2, g2s = _combo1(y, dinv8, b4e0, sk1)
        sk2 = feval(g2s)
        y3, g3s = _combo2(y, dinv8, b4e0, sk1, sk2)
        sk3 = feval(g3s)
        y4, g4s = _combo3(y, dinv8, b4e0, sk1, sk2, sk3)
        sk4 = feval(g4s)
        yn, gn = _combo4(y, dinv8, b4e0, sk1, sk2, sk3, sk4)
        return (yn, gn), yn[:, 0]

    (_, _), ys = lax.scan(step, (y8, g8), None, length=_H - 1)

    preds = jnp.concatenate([y0_8[None, :, 0], ys], axis=0)
    preds = preds.reshape(_H, _B, _NBIN)[:, :, :_N]
    return jnp.transpose(preds, (1, 2, 0))


# speedup vs baseline: 15.4337x; 1.0004x over previous
import functools

import jax
import jax.numpy as jnp
from jax import lax
from jax.experimental import pallas as pl
from jax.experimental.pallas import tpu as pltpu
from jax.experimental.pallas import tpu_sc as plsc

_B = 2
_N = 10000
_NT = _B * _N
_E = 160000
_EB = _E + _N
_H = 10
_DT = 10.0 / 9.0

_NSC = 16
_CK = 128
_CHT = 84
_EPT = _CHT * _CK
_NBIN = _NSC * 632
_SPT = 632
_NTP = _B * _NBIN

_R = 2528
_G = _NTP // _R


def _make_prop(width):
    mesh = plsc.VectorSubcoreMesh(core_axis_name="c", subcore_axis_name="s")
    nb = 4
    ng = _CHT // nb

    @functools.partial(
        pl.kernel,
        out_type=jax.ShapeDtypeStruct((_NTP, width), jnp.float32),
        mesh=mesh,
        scratch_types=[
            pltpu.VMEM((_CHT, _CK), jnp.int32),
            pltpu.VMEM((_CHT, _CK), jnp.int32),
            pltpu.VMEM((nb, _CK, width), jnp.float32),
            pltpu.VMEM_SHARED((_NBIN, width), jnp.float32),
        ] + [pltpu.SemaphoreType.DMA] * (2 * nb),
        compiler_params=pltpu.CompilerParams(use_tc_tiling_on_sc=False),
    )
    def prop(g_hbm, src_hbm, dst_hbm, zeros_hbm, out_hbm,
             srcv, dstv, rows, acc, *sems):
        gsems, ssems = sems[:nb], sems[nb:]
        c = lax.axis_index("c")
        s = lax.axis_index("s")
        pltpu.sync_copy(src_hbm.at[c, s], srcv)
        pltpu.sync_copy(dst_hbm.at[c, s], dstv)
        z0 = s * _SPT
        pltpu.sync_copy(zeros_hbm.at[pl.ds(z0, _SPT)], acc.at[pl.ds(z0, _SPT)])
        plsc.subcore_barrier()

        for b in range(nb):
            pltpu.async_copy(g_hbm.at[srcv.at[b]], rows.at[b], gsems[b])

        def body(g, carry):
            for b in range(nb):
                j = g * nb + b
                pltpu.make_async_copy(g_hbm.at[srcv.at[j]], rows.at[b],
                                      gsems[b]).wait()
                pltpu.async_copy(rows.at[b], acc.at[dstv.at[j]], ssems[b],
                                 add=True)
            for b in range(nb):
                j = g * nb + b
                pltpu.make_async_copy(rows.at[b], acc.at[dstv.at[j]],
                                      ssems[b]).wait()
                jn = j + nb

                @pl.when(jn < _CHT)
                def _():
                    pltpu.async_copy(g_hbm.at[srcv.at[jn]], rows.at[b],
                                     gsems[b])
            return carry

        lax.fori_loop(0, ng, body, 0)
        plsc.subcore_barrier()
        r0 = s * _SPT
        pltpu.sync_copy(acc.at[pl.ds(r0, _SPT)],
                        out_hbm.at[pl.ds(c * _NBIN + r0, _SPT)])

    return prop


_prop64 = _make_prop(64)
_prop8 = _make_prop(8)


def _wide_spec():
    return pl.BlockSpec((_R, 64), lambda i: (i, 0))


def _narrow_spec():
    return pl.BlockSpec((_R, 8), lambda i: (i, 0))


def _full_spec(shape):
    return pl.BlockSpec(shape, lambda i: tuple(0 for _ in shape))


def _dinv_body(deg_ref, o_ref):
    d = deg_ref[...]
    o_ref[...] = jnp.where(d > 0.0, lax.rsqrt(d), 0.0)


def _tc_dinv(deg8):
    return pl.pallas_call(
        _dinv_body,
        grid=(_G,),
        in_specs=[_narrow_spec()],
        out_specs=_narrow_spec(),
        out_shape=jax.ShapeDtypeStruct((_NTP, 8), jnp.float32),
    )(deg8)


def _l1_body(s_ref, d_ref, w0_ref, b0_ref, w1_ref, o_ref):
    dcol = d_ref[:, 0:1]
    z = s_ref[:, 0:1] * dcol
    h = jnp.tanh(z * w0_ref[...] + b0_ref[...])
    o_ref[...] = jnp.dot(h, w1_ref[...], precision=lax.Precision.HIGHEST,
                         preferred_element_type=jnp.float32) * dcol


def _tc_l1(s08, dinv8, w0, b0r, w1):
    return pl.pallas_call(
        _l1_body,
        grid=(_G,),
        in_specs=[_narrow_spec(), _narrow_spec(), _full_spec((1, 64)),
                  _full_spec((1, 64)), _full_spec((64, 64))],
        out_specs=_wide_spec(),
        out_shape=jax.ShapeDtypeStruct((_NTP, 64), jnp.float32),
    )(s08, dinv8, w0, b0r, w1)


def _mid_body(s_ref, d_ref, bp_ref, w_ref, o_ref):
    dcol = d_ref[:, 0:1]
    h = jnp.tanh(s_ref[...] * dcol + bp_ref[...])
    o_ref[...] = jnp.dot(h, w_ref[...], precision=lax.Precision.HIGHEST,
                         preferred_element_type=jnp.float32) * dcol


def _tc_mid(s, dinv8, bpr, w):
    return pl.pallas_call(
        _mid_body,
        grid=(_G,),
        in_specs=[_wide_spec(), _narrow_spec(), _full_spec((1, 64)),
                  _full_spec((64, 64))],
        out_specs=_wide_spec(),
        out_shape=jax.ShapeDtypeStruct((_NTP, 64), jnp.float32),
    )(s, dinv8, bpr, w)


def _l4_body(s_ref, d_ref, b3_ref, w4_ref, e0_ref, o_ref):
    dcol = d_ref[:, 0:1]
    h = jnp.tanh(s_ref[...] * dcol + b3_ref[...])
    v = jnp.dot(h, w4_ref[...], precision=lax.Precision.HIGHEST,
                preferred_element_type=jnp.float32) * dcol
    o_ref[...] = v * e0_ref[...]


def _tc_l4(s3, dinv8, b3r, w4, e0):
    return pl.pallas_call(
        _l4_body,
        grid=(_G,),
        in_specs=[_wide_spec(), _narrow_spec(), _full_spec((1, 64)),
                  _full_spec((64, 1)), _full_spec((1, 8))],
        out_specs=_narrow_spec(),
        out_shape=jax.ShapeDtypeStruct((_NTP, 8), jnp.float32),
    )(s3, dinv8, b3r, w4, e0)


def _make_combo(coefs):
    nk = len(coefs)

    def body(*refs):
        y_ref, d_ref, be_ref = refs[0], refs[1], refs[2]
        s_refs = refs[3:3 + nk]
        y_o, g_o = refs[3 + nk], refs[4 + nk]
        d = d_ref[...]
        acc = y_ref[...]
        for a, sref in zip(coefs, s_refs):
            acc = acc + a * (d * sref[...] + be_ref[...])
        y_o[...] = acc
        g_o[...] = acc * d

    def call(y8, dinv8, b4e0, *ss):
        return pl.pallas_call(
            body,
            grid=(_G,),
            in_specs=[_narrow_spec(), _narrow_spec(), _full_spec((1, 8))]
            + [_narrow_spec()] * nk,
            out_specs=(_narrow_spec(), _narrow_spec()),
            out_shape=(jax.ShapeDtypeStruct((_NTP, 8), jnp.float32),
                       jax.ShapeDtypeStruct((_NTP, 8), jnp.float32)),
        )(y8, dinv8, b4e0, *ss)

    return call


_combo0 = _make_combo(())
_combo1 = _make_combo((_DT / 3.0,))
_combo2 = _make_combo((-_DT / 3.0, _DT))
_combo3 = _make_combo((_DT, -_DT, _DT))
_combo4 = _make_combo((_DT / 8.0, 3.0 * _DT / 8.0, 3.0 * _DT / 8.0, _DT / 8.0))


def kernel(x, edge_index, W0, b0, W1, b1, W2, b2, W3, b3, W4, b4):
    f32 = jnp.float32
    loops = jnp.arange(_N, dtype=jnp.int32)
    srcb = jnp.concatenate([edge_index[0].astype(jnp.int32), loops])
    dstb = jnp.concatenate([edge_index[1].astype(jnp.int32), loops])
    pad = _NSC * _EPT - _EB
    srcp = jnp.concatenate([srcb, jnp.zeros((pad,), jnp.int32)])
    dstp = jnp.concatenate([dstb, jnp.full((pad,), _N, jnp.int32)])
    src_t = jnp.stack([srcp.reshape(_NSC, _CHT, _CK),
                       (srcp + _NBIN).reshape(_NSC, _CHT, _CK)])
    dst_loc = dstp.reshape(_NSC, _CHT, _CK)
    dst_t = jnp.stack([dst_loc, dst_loc])

    zeros64 = jnp.zeros((_NBIN, 64), f32)
    zeros8 = jnp.zeros((_NBIN, 8), f32)
    e0 = jnp.zeros((1, 8), f32).at[0, 0].set(1.0)
    ones8 = jnp.tile(e0, (_NTP, 1))

    b0r = b0.reshape(1, 64)
    b1r = b1.reshape(1, 64)
    b2r = b2.reshape(1, 64)
    b3r = b3.reshape(1, 64)
    w0 = W0.reshape(1, 64)
    b4e0 = b4.reshape(1, 1) * e0

    deg8 = _prop8(ones8, src_t, dst_t, zeros8)
    dinv8 = _tc_dinv(deg8)

    y0 = x[:, :, -1].astype(f32)
    y0p = jnp.pad(y0, ((0, 0), (0, _NBIN - _N))).reshape(_NTP, 1)
    y0_8 = jnp.pad(y0p, ((0, 0), (0, 7)))
    y8, g8 = _combo0(y0_8, dinv8, b4e0)

    def feval(g08):
        s0 = _prop8(g08, src_t, dst_t, zeros8)
        g1 = _tc_l1(s0, dinv8, w0, b0r, W1)
        s1 = _prop64(g1, src_t, dst_t, zeros64)
        g2 = _tc_mid(s1, dinv8, b1r, W2)
        s2 = _prop64(g2, src_t, dst_t, zeros64)
        g3 = _tc_mid(s2, dinv8, b2r, W3)
        s3 = _prop64(g3, src_t, dst_t, zeros64)
        return _tc_l4(s3, dinv8, b3r, W4, e0)

    def step(carry, _):
        y, g = carry
        sk1 = feval(g)
        y2, g2s = _combo1(y, dinv8, b4e0, sk1)
        sk2 = feval(g2s)
        y3, g3s = _combo2(y, dinv8, b4e0, sk1, sk2)
        sk3 = feval(g3s)
        y4, g4s = _combo3(y, dinv8, b4e0, sk1, sk2, sk3)
        sk4 = feval(g4s)
        yn, gn = _combo4(y, dinv8, b4e0, sk1, sk2, sk3, sk4)
        return (yn, gn), yn[:, 0]

    (_, _), ys = lax.scan(step, (y8, g8), None, length=_H - 1)

    preds = jnp.concatenate([y0_8[None, :, 0], ys], axis=0)
    preds = preds.reshape(_H, _B, _NBIN)[:, :, :_N]
    return jnp.transpose(preds, (1, 2, 0))
